# baseline (speedup 1.0000x reference)
import functools

import jax
import jax.numpy as jnp
from jax import lax
from jax.experimental import pallas as pl
from jax.experimental.pallas import tpu as pltpu
from jax.experimental.pallas import tpu_sc as plsc

NC = 2
NS = 16
NW = NC * NS
L = 16

CHUNK = 128


def _sc_body(B, D, b_per_w,
             user_idx_hbm, item_idx_hbm, know_hbm,
             theta_tab_hbm, a_tab_hbm, b_tab_hbm,
             pred_out, theta_out, a_out, b_out,
             uidx_v, iidx_v, uidx_h, iidx_h,
             theta_s0, theta_s1, a_s0, a_s1, know_s0, know_s1,
             t_c0, t_c1, a_c0, a_c1, b_v, pred_v,
             sem_g0, sem_g1, sem_o0, sem_o1):
    n_chunks = b_per_w // CHUNK
    wid = lax.axis_index("s") * NC + lax.axis_index("c")
    base = pl.multiple_of(wid * b_per_w, b_per_w)
    hbase = pl.multiple_of(wid * (b_per_w // 2), b_per_w // 2)

    pltpu.sync_copy(user_idx_hbm.at[pl.ds(base, b_per_w)], uidx_v)
    pltpu.sync_copy(item_idx_hbm.at[pl.ds(base, b_per_w)], iidx_v)
    for k in range(b_per_w // L):
        sl = pl.ds(k * L, L)
        uidx_h[sl] = lax.shift_right_logical(uidx_v[sl], 1)
        iidx_h[sl] = lax.shift_right_logical(iidx_v[sl], 1)

    theta_s = (theta_s0, theta_s1)
    a_s = (a_s0, a_s1)
    know_s = (know_s0, know_s1)
    t_c = (t_c0, t_c1)
    a_c = (a_c0, a_c1)
    sem_g = (sem_g0, sem_g1)
    sem_o = (sem_o0, sem_o1)

    def fire_gathers(cix, b):
        isl = pl.ds(cix * CHUNK, CHUNK)
        pltpu.async_copy(theta_tab_hbm.at[uidx_h.at[isl]], theta_s[b],
                         sem_g[b])
        pltpu.async_copy(a_tab_hbm.at[iidx_h.at[isl]], a_s[b], sem_g[b])
        pltpu.async_copy(b_tab_hbm.at[iidx_v.at[isl]], b_v.at[isl], sem_g[b])
        pltpu.async_copy(
            know_hbm.at[pl.ds(hbase + cix * (CHUNK // 2), CHUNK // 2)],
            know_s[b], sem_g[b])

    def wait_gathers(cix, b):
        isl = pl.ds(cix * CHUNK, CHUNK)
        pltpu.make_async_copy(theta_tab_hbm.at[uidx_h.at[isl]], theta_s[b],
                              sem_g[b]).wait()
        pltpu.make_async_copy(a_tab_hbm.at[iidx_h.at[isl]], a_s[b],
                              sem_g[b]).wait()
        pltpu.make_async_copy(b_tab_hbm.at[iidx_v.at[isl]], b_v.at[isl],
                              sem_g[b]).wait()
        pltpu.make_async_copy(
            know_hbm.at[pl.ds(hbase + cix * (CHUNK // 2), CHUNK // 2)],
            know_s[b], sem_g[b]).wait()

    def fire_out(cix, b):
        osl = pl.ds(hbase + cix * (CHUNK // 2), CHUNK // 2)
        pltpu.async_copy(t_c[b], theta_out.at[osl], sem_o[b])
        pltpu.async_copy(a_c[b], a_out.at[osl], sem_o[b])

    def wait_out(cix, b):
        osl = pl.ds(hbase + cix * (CHUNK // 2), CHUNK // 2)
        pltpu.make_async_copy(t_c[b], theta_out.at[osl], sem_o[b]).wait()
        pltpu.make_async_copy(a_c[b], a_out.at[osl], sem_o[b]).wait()

    lane = lax.iota(jnp.int32, L)
    n_sub = D // L

    def make_group_body(cix, b):
        def group_body(g, carry):
            gb = cix * CHUNK + g * L
            u16 = uidx_v[pl.ds(gb, L)]
            i16 = iidx_v[pl.ds(gb, L)]
            toff16 = lax.shift_left(jnp.bitwise_and(u16, 1), 6)
            aoff16 = lax.shift_left(jnp.bitwise_and(i16, 1), 6)
            zvec = jnp.zeros((L,), jnp.float32)
            for rl in range(L):
                r_loc = g * L + rl
                toff = toff16[rl]
                aoff = aoff16[rl]
                koff = (rl & 1) * D
                acc = None
                for c in range(n_sub):
                    tv = theta_s[b][r_loc, pl.ds(toff + c * L, L)]
                    av = a_s[b][r_loc, pl.ds(aoff + c * L, L)]
                    khalf = pl.ds(koff + c * L, L)
                    kv = know_s[b][g * (L // 2) + rl // 2, khalf]
                    t_c[b][g * (L // 2) + rl // 2, khalf] = tv
                    a_c[b][g * (L // 2) + rl // 2, khalf] = av
                    prod = tv * av * kv
                    acc = prod if acc is None else acc + prod
                zvec = jnp.where(lane == rl, jnp.sum(acc), zvec)
            z = zvec - b_v[pl.ds(gb, L)]
            pred_v[pl.ds(gb, L)] = 1.0 / (1.0 + jnp.exp(-z))
            return carry
        return group_body

    fire_gathers(0, 0)
    for cix in range(n_chunks):
        b = cix % 2
        wait_gathers(cix, b)
        if cix + 1 < n_chunks:
            fire_gathers(cix + 1, 1 - b)
        if cix >= 2:
            wait_out(cix - 2, b)
        lax.fori_loop(0, CHUNK // L, make_group_body(cix, b), 0)
        fire_out(cix, b)

    wait_out(n_chunks - 2, n_chunks % 2)
    wait_out(n_chunks - 1, 1 - n_chunks % 2)

    pltpu.async_copy(pred_v, pred_out.at[pl.ds(base, b_per_w)], sem_o0)
    pltpu.async_copy(b_v, b_out.at[pl.ds(base, b_per_w)], sem_o0)
    pltpu.make_async_copy(pred_v, pred_out.at[pl.ds(base, b_per_w)],
                          sem_o0).wait()
    pltpu.make_async_copy(b_v, b_out.at[pl.ds(base, b_per_w)],
                          sem_o0).wait()


@jax.jit
def _emb_icd(user_idx, item_idx, know, theta_table, a_table, b_table):
    B, D = know.shape
    assert B % (NW * CHUNK) == 0 and D % L == 0
    b_per_w = B // NW

    mesh = plsc.VectorSubcoreMesh(core_axis_name="c", subcore_axis_name="s",
                                  num_cores=NC, num_subcores=NS)
    fn = pl.kernel(
        functools.partial(_sc_body, B, D, b_per_w),
        out_type=(
            jax.ShapeDtypeStruct((B,), jnp.float32),
            jax.ShapeDtypeStruct((B // 2, 2 * D), jnp.float32),
            jax.ShapeDtypeStruct((B // 2, 2 * D), jnp.float32),
            jax.ShapeDtypeStruct((B,), jnp.float32),
        ),
        mesh=mesh,
        scratch_types=[
            pltpu.VMEM((b_per_w,), jnp.int32),
            pltpu.VMEM((b_per_w,), jnp.int32),
            pltpu.VMEM((b_per_w,), jnp.int32),
            pltpu.VMEM((b_per_w,), jnp.int32),
            pltpu.VMEM((CHUNK, 2 * D), jnp.float32),
            pltpu.VMEM((CHUNK, 2 * D), jnp.float32),
            pltpu.VMEM((CHUNK, 2 * D), jnp.float32),
            pltpu.VMEM((CHUNK, 2 * D), jnp.float32),
            pltpu.VMEM((CHUNK // 2, 2 * D), jnp.float32),
            pltpu.VMEM((CHUNK // 2, 2 * D), jnp.float32),
            pltpu.VMEM((CHUNK // 2, 2 * D), jnp.float32),
            pltpu.VMEM((CHUNK // 2, 2 * D), jnp.float32),
            pltpu.VMEM((CHUNK // 2, 2 * D), jnp.float32),
            pltpu.VMEM((CHUNK // 2, 2 * D), jnp.float32),
            pltpu.VMEM((b_per_w,), jnp.float32),
            pltpu.VMEM((b_per_w,), jnp.float32),
            pltpu.SemaphoreType.DMA,
            pltpu.SemaphoreType.DMA,
            pltpu.SemaphoreType.DMA,
            pltpu.SemaphoreType.DMA,
        ],
        compiler_params=pltpu.CompilerParams(needs_layout_passes=False,
                                             use_tc_tiling_on_sc=True),
        name="emb_icd_sc",
    )
    pred, theta_p, a_p, b_flat = fn(
        user_idx, item_idx, know.reshape(B // 2, 2 * D),
        theta_table.reshape(-1, 2 * D), a_table.reshape(-1, 2 * D),
        b_table.reshape(-1))
    return (pred, theta_p.reshape(B, D), a_p.reshape(B, D), b_flat)


def kernel(user_idx, item_idx, know, theta_table, a_table, b_table):
    user_idx = user_idx.astype(jnp.int32)
    item_idx = item_idx.astype(jnp.int32)
    pred, theta, a, b_flat = _emb_icd(user_idx, item_idx, know,
                                      theta_table, a_table, b_table)
    return (pred, theta, a, b_flat.reshape(-1, 1))


# speedup vs baseline: 1.9015x; 1.9015x over previous
import functools

import jax
import jax.numpy as jnp
from jax import lax
from jax.experimental import pallas as pl
from jax.experimental.pallas import tpu as pltpu
from jax.experimental.pallas import tpu_sc as plsc

NC = 2
NS = 16
NW = NC * NS
L = 16
TS = 8

CHUNK = 16
HCOLS = 256


def _sc_body(B, D, b_per_w,
             user_idx_hbm, item_idx_hbm, know_hbm,
             theta_tab_hbm, a_tab_hbm, b_tab_hbm,
             pred_out, theta_t_out, a_t_out, b_out,
             uidx_v, iidx_v,
             theta_p0, theta_p1, a_p0, a_p1, know_p0, know_p1,
             t_T, a_T, b_v, pred_v,
             sem_g0, sem_g1, sem_o):
    n_chunks = b_per_w // CHUNK
    wid = lax.axis_index("s") * NC + lax.axis_index("c")
    base = pl.multiple_of(wid * b_per_w, b_per_w)
    kbase = pl.multiple_of(wid * b_per_w * D, b_per_w * D)

    pltpu.sync_copy(user_idx_hbm.at[pl.ds(base, b_per_w)], uidx_v)
    pltpu.sync_copy(item_idx_hbm.at[pl.ds(base, b_per_w)], iidx_v)

    theta_p = (theta_p0, theta_p1)
    a_p = (a_p0, a_p1)
    know_p = (know_p0, know_p1)
    sem_g = (sem_g0, sem_g1)

    def fire_gathers(cix, b):
        isl = pl.ds(cix * CHUNK, CHUNK)
        u16 = uidx_v[isl]
        i16 = iidx_v[isl]
        ut = lax.shift_right_logical(u16, 3)
        it = lax.shift_right_logical(i16, 3)
        for rl in range(CHUNK):
            pltpu.async_copy(theta_tab_hbm.at[ut[rl]], theta_p[b].at[rl],
                             sem_g[b])
            pltpu.async_copy(a_tab_hbm.at[it[rl]], a_p[b].at[rl], sem_g[b])
        pltpu.async_copy(b_tab_hbm.at[iidx_v.at[isl]], b_v.at[isl], sem_g[b])
        pltpu.async_copy(know_hbm.at[pl.ds(kbase + cix * (CHUNK * D),
                                           CHUNK * D)],
                         know_p[b], sem_g[b])

    def wait_gathers(cix, b):
        isl = pl.ds(cix * CHUNK, CHUNK)
        for rl in range(CHUNK):
            pltpu.make_async_copy(theta_tab_hbm.at[0], theta_p[b].at[rl],
                                  sem_g[b]).wait()
            pltpu.make_async_copy(a_tab_hbm.at[0], a_p[b].at[rl],
                                  sem_g[b]).wait()
        pltpu.make_async_copy(b_tab_hbm.at[iidx_v.at[isl]], b_v.at[isl],
                              sem_g[b]).wait()
        pltpu.make_async_copy(know_hbm.at[pl.ds(kbase + cix * (CHUNK * D),
                                                CHUNK * D)],
                              know_p[b], sem_g[b]).wait()

    def flush_outputs(col0):
        osl = pl.ds(base + col0, HCOLS)
        pltpu.async_copy(t_T, theta_t_out.at[:, osl], sem_o)
        pltpu.async_copy(a_T, a_t_out.at[:, osl], sem_o)
        pltpu.make_async_copy(t_T, theta_t_out.at[:, osl], sem_o).wait()
        pltpu.make_async_copy(a_T, a_t_out.at[:, osl], sem_o).wait()

    lane = lax.iota(jnp.int32, L)
    n_sub = D // L

    def compute(jj, cix, b):
        isl = pl.ds(cix * CHUNK, CHUNK)
        u16 = uidx_v[isl]
        i16 = iidx_v[isl]
        su16 = jnp.bitwise_and(u16, TS - 1)
        si16 = jnp.bitwise_and(i16, TS - 1)
        colb = cix * CHUNK - jnp.where(jj >= (n_chunks // 4), HCOLS, 0)
        zvec = jnp.zeros((L,), jnp.float32)
        for rl in range(CHUNK):
            su = su16[rl]
            si = si16[rl]
            cvec = jnp.full((L,), 0, jnp.int32) + (colb + rl)
            acc = None
            for c in range(n_sub):
                csl = pl.ds(c * L, L)
                tv = theta_p[b][rl, su, csl]
                av = a_p[b][rl, si, csl]
                kv = know_p[b][pl.ds(rl * D + c * L, L)]
                dvec = lane + c * L
                plsc.store_scatter(t_T, [dvec, cvec], tv)
                plsc.store_scatter(a_T, [dvec, cvec], av)
                prod = tv * av * kv
                acc = prod if acc is None else acc + prod
            zvec = jnp.where(lane == rl, jnp.sum(acc), zvec)
        z = zvec - b_v[isl]
        pred_v[isl] = 1.0 / (1.0 + jnp.exp(-z))

    fire_gathers(0, 0)

    def loop_body(jj, carry):
        for b in (0, 1):
            cix = jj * 2 + b
            wait_gathers(cix, b)
            nxt = jnp.minimum(cix + 1, n_chunks - 1)
            fire_gathers(nxt, 1 - b)
            if b == 0:
                @pl.when(jj == n_chunks // 4)
                def _():
                    flush_outputs(0)
            compute(jj, cix, b)
        return carry

    lax.fori_loop(0, n_chunks // 2, loop_body, 0)

    wait_gathers(n_chunks - 1, 0)
    flush_outputs(HCOLS)

    pltpu.async_copy(pred_v, pred_out.at[pl.ds(base, b_per_w)], sem_o)
    pltpu.async_copy(b_v, b_out.at[pl.ds(base, b_per_w)], sem_o)
    pltpu.make_async_copy(pred_v, pred_out.at[pl.ds(base, b_per_w)],
                          sem_o).wait()
    pltpu.make_async_copy(b_v, b_out.at[pl.ds(base, b_per_w)],
                          sem_o).wait()


@jax.jit
def _emb_icd(user_idx, item_idx, know, theta_table, a_table, b_table):
    B, D = know.shape
    assert B % (NW * CHUNK) == 0 and D % L == 0
    b_per_w = B // NW

    mesh = plsc.VectorSubcoreMesh(core_axis_name="c", subcore_axis_name="s",
                                  num_cores=NC, num_subcores=NS)
    fn = pl.kernel(
        functools.partial(_sc_body, B, D, b_per_w),
        out_type=(
            jax.ShapeDtypeStruct((B,), jnp.float32),
            jax.ShapeDtypeStruct((D, B), jnp.float32),
            jax.ShapeDtypeStruct((D, B), jnp.float32),
            jax.ShapeDtypeStruct((B,), jnp.float32),
        ),
        mesh=mesh,
        scratch_types=[
            pltpu.VMEM((b_per_w,), jnp.int32),
            pltpu.VMEM((b_per_w,), jnp.int32),
            pltpu.VMEM((CHUNK, TS, D), jnp.float32),
            pltpu.VMEM((CHUNK, TS, D), jnp.float32),
            pltpu.VMEM((CHUNK, TS, D), jnp.float32),
            pltpu.VMEM((CHUNK, TS, D), jnp.float32),
            pltpu.VMEM((CHUNK * D,), jnp.float32),
            pltpu.VMEM((CHUNK * D,), jnp.float32),
            pltpu.VMEM((D, HCOLS), jnp.float32),
            pltpu.VMEM((D, HCOLS), jnp.float32),
            pltpu.VMEM((b_per_w,), jnp.float32),
            pltpu.VMEM((b_per_w,), jnp.float32),
            pltpu.SemaphoreType.DMA,
            pltpu.SemaphoreType.DMA,
            pltpu.SemaphoreType.DMA,
        ],
        compiler_params=pltpu.CompilerParams(needs_layout_passes=False,
                                             use_tc_tiling_on_sc=True),
        name="emb_icd_sc",
    )
    theta_tiles = theta_table.reshape(-1, TS, D)
    a_tiles = a_table.reshape(-1, TS, D)
    pred, theta_t, a_t, b_flat = fn(
        user_idx, item_idx, know.reshape(-1), theta_tiles, a_tiles,
        b_table.reshape(-1))
    return (pred, theta_t.T, a_t.T, b_flat)


def kernel(user_idx, item_idx, know, theta_table, a_table, b_table):
    user_idx = user_idx.astype(jnp.int32)
    item_idx = item_idx.astype(jnp.int32)
    pred, theta, a, b_flat = _emb_icd(user_idx, item_idx, know,
                                      theta_table, a_table, b_table)
    return (pred, theta, a, b_flat.reshape(-1, 1))


# speedup vs baseline: 1.9976x; 1.0505x over previous
import functools

import jax
import jax.numpy as jnp
from jax import lax
from jax.experimental import pallas as pl
from jax.experimental.pallas import tpu as pltpu
from jax.experimental.pallas import tpu_sc as plsc

NC = 2
NS = 16
NW = NC * NS
L = 16
TS = 8

CHUNK = 16
HCOLS = 256


def _sc_body(B, D, b_per_w,
             user_idx_hbm, item_idx_hbm, know_hbm,
             theta_tab_hbm, a_tab_hbm, b_tab_hbm,
             pred_out, theta_t_out, a_t_out, b_out,
             uidx_v, iidx_v,
             theta_p0, theta_p1, a_p0, a_p1, know_p0, know_p1,
             t_T, a_T, b_v, pred_v,
             sem_g0, sem_g1, sem_o):
    n_chunks = b_per_w // CHUNK
    wid = lax.axis_index("s") * NC + lax.axis_index("c")
    base = pl.multiple_of(wid * b_per_w, b_per_w)
    kbase = pl.multiple_of(wid * b_per_w * D, b_per_w * D)

    pltpu.sync_copy(user_idx_hbm.at[pl.ds(base, b_per_w)], uidx_v)
    pltpu.sync_copy(item_idx_hbm.at[pl.ds(base, b_per_w)], iidx_v)

    theta_p = (theta_p0, theta_p1)
    a_p = (a_p0, a_p1)
    know_p = (know_p0, know_p1)
    sem_g = (sem_g0, sem_g1)

    def fire_gathers(cix, b):
        isl = pl.ds(cix * CHUNK, CHUNK)
        u16 = uidx_v[isl]
        i16 = iidx_v[isl]
        ut = lax.shift_right_logical(u16, 3)
        it = lax.shift_right_logical(i16, 3)
        for rl in range(CHUNK):
            pltpu.async_copy(theta_tab_hbm.at[ut[rl]], theta_p[b].at[rl],
                             sem_g[b])
            pltpu.async_copy(a_tab_hbm.at[it[rl]], a_p[b].at[rl], sem_g[b])
        pltpu.async_copy(b_tab_hbm.at[iidx_v.at[isl]], b_v.at[isl], sem_g[b])
        pltpu.async_copy(know_hbm.at[pl.ds(kbase + cix * (CHUNK * D),
                                           CHUNK * D)],
                         know_p[b], sem_g[b])

    def wait_gathers(cix, b):
        isl = pl.ds(cix * CHUNK, CHUNK)
        pltpu.make_async_copy(theta_tab_hbm.at[pl.ds(0, CHUNK)], theta_p[b],
                              sem_g[b]).wait()
        pltpu.make_async_copy(a_tab_hbm.at[pl.ds(0, CHUNK)], a_p[b],
                              sem_g[b]).wait()
        pltpu.make_async_copy(b_tab_hbm.at[iidx_v.at[isl]], b_v.at[isl],
                              sem_g[b]).wait()
        pltpu.make_async_copy(know_hbm.at[pl.ds(kbase + cix * (CHUNK * D),
                                                CHUNK * D)],
                              know_p[b], sem_g[b]).wait()

    def flush_outputs(col0):
        osl = pl.ds(base + col0, HCOLS)
        pltpu.async_copy(t_T, theta_t_out.at[:, osl], sem_o)
        pltpu.async_copy(a_T, a_t_out.at[:, osl], sem_o)
        pltpu.make_async_copy(t_T, theta_t_out.at[:, osl], sem_o).wait()
        pltpu.make_async_copy(a_T, a_t_out.at[:, osl], sem_o).wait()

    lane = lax.iota(jnp.int32, L)
    n_sub = D // L

    def compute(jj, cix, b):
        isl = pl.ds(cix * CHUNK, CHUNK)
        u16 = uidx_v[isl]
        i16 = iidx_v[isl]
        su16 = jnp.bitwise_and(u16, TS - 1)
        si16 = jnp.bitwise_and(i16, TS - 1)
        colb = cix * CHUNK - jnp.where(jj >= (n_chunks // 4), HCOLS, 0)
        zvec = jnp.zeros((L,), jnp.float32)
        for rl in range(CHUNK):
            su = su16[rl]
            si = si16[rl]
            cvec = jnp.full((L,), 0, jnp.int32) + (colb + rl)
            acc = None
            for c in range(n_sub):
                csl = pl.ds(c * L, L)
                tv = theta_p[b][rl, su, csl]
                av = a_p[b][rl, si, csl]
                kv = know_p[b][pl.ds(rl * D + c * L, L)]
                dvec = lane + c * L
                plsc.store_scatter(t_T, [dvec, cvec], tv)
                plsc.store_scatter(a_T, [dvec, cvec], av)
                prod = tv * av * kv
                acc = prod if acc is None else acc + prod
            zvec = jnp.where(lane == rl, jnp.sum(acc), zvec)
        z = zvec - b_v[isl]
        pred_v[isl] = 1.0 / (1.0 + jnp.exp(-z))

    fire_gathers(0, 0)

    def loop_body(jj, carry):
        for b in (0, 1):
            cix = jj * 2 + b
            nxt = jnp.minimum(cix + 1, n_chunks - 1)
            fire_gathers(nxt, 1 - b)
            wait_gathers(cix, b)
            if b == 0:
                @pl.when(jj == n_chunks // 4)
                def _():
                    flush_outputs(0)
            compute(jj, cix, b)
        return carry

    lax.fori_loop(0, n_chunks // 2, loop_body, 0)

    wait_gathers(n_chunks - 1, 0)
    flush_outputs(HCOLS)

    pltpu.async_copy(pred_v, pred_out.at[pl.ds(base, b_per_w)], sem_o)
    pltpu.async_copy(b_v, b_out.at[pl.ds(base, b_per_w)], sem_o)
    pltpu.make_async_copy(pred_v, pred_out.at[pl.ds(base, b_per_w)],
                          sem_o).wait()
    pltpu.make_async_copy(b_v, b_out.at[pl.ds(base, b_per_w)],
                          sem_o).wait()


@jax.jit
def _emb_icd(user_idx, item_idx, know, theta_table, a_table, b_table):
    B, D = know.shape
    assert B % (NW * CHUNK) == 0 and D % L == 0
    b_per_w = B // NW

    mesh = plsc.VectorSubcoreMesh(core_axis_name="c", subcore_axis_name="s",
                                  num_cores=NC, num_subcores=NS)
    fn = pl.kernel(
        functools.partial(_sc_body, B, D, b_per_w),
        out_type=(
            jax.ShapeDtypeStruct((B,), jnp.float32),
            jax.ShapeDtypeStruct((D, B), jnp.float32),
            jax.ShapeDtypeStruct((D, B), jnp.float32),
            jax.ShapeDtypeStruct((B,), jnp.float32),
        ),
        mesh=mesh,
        scratch_types=[
            pltpu.VMEM((b_per_w,), jnp.int32),
            pltpu.VMEM((b_per_w,), jnp.int32),
            pltpu.VMEM((CHUNK, TS, D), jnp.float32),
            pltpu.VMEM((CHUNK, TS, D), jnp.float32),
            pltpu.VMEM((CHUNK, TS, D), jnp.float32),
            pltpu.VMEM((CHUNK, TS, D), jnp.float32),
            pltpu.VMEM((CHUNK * D,), jnp.float32),
            pltpu.VMEM((CHUNK * D,), jnp.float32),
            pltpu.VMEM((D, HCOLS), jnp.float32),
            pltpu.VMEM((D, HCOLS), jnp.float32),
            pltpu.VMEM((b_per_w,), jnp.float32),
            pltpu.VMEM((b_per_w,), jnp.float32),
            pltpu.SemaphoreType.DMA,
            pltpu.SemaphoreType.DMA,
            pltpu.SemaphoreType.DMA,
        ],
        compiler_params=pltpu.CompilerParams(needs_layout_passes=False,
                                             use_tc_tiling_on_sc=True),
        name="emb_icd_sc",
    )
    theta_tiles = theta_table.reshape(-1, TS, D)
    a_tiles = a_table.reshape(-1, TS, D)
    pred, theta_t, a_t, b_flat = fn(
        user_idx, item_idx, know.reshape(-1), theta_tiles, a_tiles,
        b_table.reshape(-1))
    return (pred, theta_t.T, a_t.T, b_flat)


def kernel(user_idx, item_idx, know, theta_table, a_table, b_table):
    user_idx = user_idx.astype(jnp.int32)
    item_idx = item_idx.astype(jnp.int32)
    pred, theta, a, b_flat = _emb_icd(user_idx, item_idx, know,
                                      theta_table, a_table, b_table)
    return (pred, theta, a, b_flat.reshape(-1, 1))


# speedup vs baseline: 1.9977x; 1.0000x over previous
import functools

import jax
import jax.numpy as jnp
from jax import lax
from jax.experimental import pallas as pl
from jax.experimental.pallas import tpu as pltpu
from jax.experimental.pallas import tpu_sc as plsc

NC = 2
NS = 16
NW = NC * NS
L = 16
TS = 8

CHUNK = 16
NBUF = 3
HCOLS = 128


def _sc_body(B, D, b_per_w,
             user_idx_hbm, item_idx_hbm, know_hbm,
             theta_tab_hbm, a_tab_hbm, b_tab_hbm,
             pred_out, theta_t_out, a_t_out, b_out,
             uidx_v, iidx_v,
             theta_p0, theta_p1, theta_p2, a_p0, a_p1, a_p2,
             know_p0, know_p1, know_p2,
             t_T, a_T, b_v, pred_v,
             sem_g0, sem_g1, sem_g2, sem_o):
    n_chunks = b_per_w // CHUNK
    wid = lax.axis_index("s") * NC + lax.axis_index("c")
    base = pl.multiple_of(wid * b_per_w, b_per_w)
    kbase = pl.multiple_of(wid * b_per_w * D, b_per_w * D)

    pltpu.sync_copy(user_idx_hbm.at[pl.ds(base, b_per_w)], uidx_v)
    pltpu.sync_copy(item_idx_hbm.at[pl.ds(base, b_per_w)], iidx_v)

    theta_p = (theta_p0, theta_p1, theta_p2)
    a_p = (a_p0, a_p1, a_p2)
    know_p = (know_p0, know_p1, know_p2)
    sem_g = (sem_g0, sem_g1, sem_g2)

    def fire_gathers(cix, b):
        isl = pl.ds(cix * CHUNK, CHUNK)
        u16 = uidx_v[isl]
        i16 = iidx_v[isl]
        ut = lax.shift_right_logical(u16, 3)
        it = lax.shift_right_logical(i16, 3)
        for rl in range(CHUNK):
            pltpu.async_copy(theta_tab_hbm.at[ut[rl]], theta_p[b].at[rl],
                             sem_g[b])
            pltpu.async_copy(a_tab_hbm.at[it[rl]], a_p[b].at[rl], sem_g[b])
        pltpu.async_copy(b_tab_hbm.at[iidx_v.at[isl]], b_v.at[isl], sem_g[b])
        pltpu.async_copy(know_hbm.at[pl.ds(kbase + cix * (CHUNK * D),
                                           CHUNK * D)],
                         know_p[b], sem_g[b])

    def wait_gathers(cix, b):
        isl = pl.ds(cix * CHUNK, CHUNK)
        pltpu.make_async_copy(theta_tab_hbm.at[pl.ds(0, CHUNK)], theta_p[b],
                              sem_g[b]).wait()
        pltpu.make_async_copy(a_tab_hbm.at[pl.ds(0, CHUNK)], a_p[b],
                              sem_g[b]).wait()
        pltpu.make_async_copy(b_tab_hbm.at[iidx_v.at[isl]], b_v.at[isl],
                              sem_g[b]).wait()
        pltpu.make_async_copy(know_hbm.at[pl.ds(kbase + cix * (CHUNK * D),
                                                CHUNK * D)],
                              know_p[b], sem_g[b]).wait()

    def flush_outputs(col0):
        osl = pl.ds(base + col0, HCOLS)
        pltpu.async_copy(t_T, theta_t_out.at[:, osl], sem_o)
        pltpu.async_copy(a_T, a_t_out.at[:, osl], sem_o)
        pltpu.make_async_copy(t_T, theta_t_out.at[:, osl], sem_o).wait()
        pltpu.make_async_copy(a_T, a_t_out.at[:, osl], sem_o).wait()

    lane = lax.iota(jnp.int32, L)
    n_sub = D // L

    def maybe_flush(cix):
        @pl.when(jnp.logical_and(jnp.equal(jnp.bitwise_and(cix, 7), 0),
                                 cix > 0))
        def _():
            col0 = pl.multiple_of(
                lax.shift_left(lax.shift_right_logical(cix, 3) - 1, 7),
                HCOLS)
            flush_outputs(col0)

    def compute(cix, b):
        isl = pl.ds(cix * CHUNK, CHUNK)
        u16 = uidx_v[isl]
        i16 = iidx_v[isl]
        su16 = jnp.bitwise_and(u16, TS - 1)
        si16 = jnp.bitwise_and(i16, TS - 1)
        colb = lax.shift_left(jnp.bitwise_and(cix, 7), 4)
        zvec = jnp.zeros((L,), jnp.float32)
        for rl in range(CHUNK):
            su = su16[rl]
            si = si16[rl]
            cvec = jnp.zeros((L,), jnp.int32) + (colb + rl)
            acc = None
            for c in range(n_sub):
                csl = pl.ds(c * L, L)
                tv = theta_p[b][rl, su, csl]
                av = a_p[b][rl, si, csl]
                kv = know_p[b][pl.ds(rl * D + c * L, L)]
                dvec = lane + c * L
                plsc.store_scatter(t_T, [dvec, cvec], tv)
                plsc.store_scatter(a_T, [dvec, cvec], av)
                prod = tv * av * kv
                acc = prod if acc is None else acc + prod
            zvec = jnp.where(lane == rl, jnp.sum(acc), zvec)
        z = zvec - b_v[isl]
        pred_v[isl] = 1.0 / (1.0 + jnp.exp(-z))

    fire_gathers(0, 0)
    fire_gathers(1, 1)

    def loop_body(jj, carry):
        for b in range(NBUF):
            cix = jj * NBUF + b
            fire_gathers(cix + 2, (b + 2) % NBUF)
            wait_gathers(cix, b)
            maybe_flush(cix)
            compute(cix, b)
        return carry

    lax.fori_loop(0, (n_chunks - 2) // NBUF, loop_body, 0)

    for cix in range(n_chunks - 2, n_chunks):
        b = cix % NBUF
        wait_gathers(cix, b)
        maybe_flush(cix)
        compute(cix, b)

    flush_outputs((n_chunks - TS) * CHUNK)

    pltpu.async_copy(pred_v, pred_out.at[pl.ds(base, b_per_w)], sem_o)
    pltpu.async_copy(b_v, b_out.at[pl.ds(base, b_per_w)], sem_o)
    pltpu.make_async_copy(pred_v, pred_out.at[pl.ds(base, b_per_w)],
                          sem_o).wait()
    pltpu.make_async_copy(b_v, b_out.at[pl.ds(base, b_per_w)],
                          sem_o).wait()


@jax.jit
def _emb_icd(user_idx, item_idx, know, theta_table, a_table, b_table):
    B, D = know.shape
    assert B % (NW * CHUNK) == 0 and D % L == 0
    b_per_w = B // NW
    assert (b_per_w // CHUNK - 2) % NBUF == 0

    mesh = plsc.VectorSubcoreMesh(core_axis_name="c", subcore_axis_name="s",
                                  num_cores=NC, num_subcores=NS)
    fn = pl.kernel(
        functools.partial(_sc_body, B, D, b_per_w),
        out_type=(
            jax.ShapeDtypeStruct((B,), jnp.float32),
            jax.ShapeDtypeStruct((D, B), jnp.float32),
            jax.ShapeDtypeStruct((D, B), jnp.float32),
            jax.ShapeDtypeStruct((B,), jnp.float32),
        ),
        mesh=mesh,
        scratch_types=[
            pltpu.VMEM((b_per_w,), jnp.int32),
            pltpu.VMEM((b_per_w,), jnp.int32),
            pltpu.VMEM((CHUNK, TS, D), jnp.float32),
            pltpu.VMEM((CHUNK, TS, D), jnp.float32),
            pltpu.VMEM((CHUNK, TS, D), jnp.float32),
            pltpu.VMEM((CHUNK, TS, D), jnp.float32),
            pltpu.VMEM((CHUNK, TS, D), jnp.float32),
            pltpu.VMEM((CHUNK, TS, D), jnp.float32),
            pltpu.VMEM((CHUNK * D,), jnp.float32),
            pltpu.VMEM((CHUNK * D,), jnp.float32),
            pltpu.VMEM((CHUNK * D,), jnp.float32),
            pltpu.VMEM((D, HCOLS), jnp.float32),
            pltpu.VMEM((D, HCOLS), jnp.float32),
            pltpu.VMEM((b_per_w,), jnp.float32),
            pltpu.VMEM((b_per_w,), jnp.float32),
            pltpu.SemaphoreType.DMA,
            pltpu.SemaphoreType.DMA,
            pltpu.SemaphoreType.DMA,
            pltpu.SemaphoreType.DMA,
        ],
        compiler_params=pltpu.CompilerParams(needs_layout_passes=False,
                                             use_tc_tiling_on_sc=True),
        name="emb_icd_sc",
    )
    theta_tiles = theta_table.reshape(-1, TS, D)
    a_tiles = a_table.reshape(-1, TS, D)
    pred, theta_t, a_t, b_flat = fn(
        user_idx, item_idx, know.reshape(-1), theta_tiles, a_tiles,
        b_table.reshape(-1))
    return (pred, theta_t.T, a_t.T, b_flat)


def kernel(user_idx, item_idx, know, theta_table, a_table, b_table):
    user_idx = user_idx.astype(jnp.int32)
    item_idx = item_idx.astype(jnp.int32)
    pred, theta, a, b_flat = _emb_icd(user_idx, item_idx, know,
                                      theta_table, a_table, b_table)
    return (pred, theta, a, b_flat.reshape(-1, 1))
